# baseline (speedup 1.0000x reference)
import functools

import jax
import jax.numpy as jnp
from jax import lax
from jax.experimental import pallas as pl
from jax.experimental.pallas import tpu as pltpu
from jax.experimental.pallas import tpu_sc as plsc

N = 10000
NP = 10240
E = 320000
D = 128
H = 128
G = 128
OUTD = 2
NC = 2
NS = 16
NW = NC * NS
L = 16
CH = 128
KT = 80
RT = NW * KT
KD = RT // NS
RPT = NP // NW
RPS = NP // NS
TRASH = N


def _mm_body(x_ref, w_ref, o_ref):
    o_ref[...] = jnp.dot(x_ref[...], w_ref[...], preferred_element_type=jnp.float32)


@jax.jit
def _matmul(x, w):
    m, k = x.shape
    n = w.shape[1]
    bm = 512
    return pl.pallas_call(
        _mm_body,
        grid=(m // bm,),
        in_specs=[
            pl.BlockSpec((bm, k), lambda i: (i, 0)),
            pl.BlockSpec((k, n), lambda i: (0, 0)),
        ],
        out_specs=pl.BlockSpec((bm, n), lambda i: (i, 0)),
        out_shape=jax.ShapeDtypeStruct((m, n), jnp.float32),
    )(x, w)


def _news_body(x_ref, b2_ref, w_ref, bias_ref, o_ref):
    rows_g = lax.broadcasted_iota(jnp.int32, (G, 1), 0)

    def cnt_step(k2, acc):
        blk8 = b2_ref[pl.ds(k2 * 8, 8), :]
        for r8 in range(8):
            row = blk8[r8:r8 + 1, :]
            acc = acc + jnp.sum((rows_g == row).astype(jnp.float32), axis=1,
                                keepdims=True)
        return acc
    cnt = lax.fori_loop(0, (NP // 128) // 8, cnt_step,
                        jnp.zeros((G, 1), jnp.float32))

    rr = lax.broadcasted_iota(jnp.int32, (G, G), 0)
    cc = lax.broadcasted_iota(jnp.int32, (G, G), 1)
    m_lt = (cc < rr).astype(jnp.float32)
    first = jnp.dot(m_lt, cnt, preferred_element_type=jnp.float32)
    first = jnp.minimum(first, float(N - 1)).astype(jnp.int32)

    bk = 256

    def gath_step(k2, acc):
        io = lax.broadcasted_iota(jnp.int32, (G, bk), 1) + k2 * bk
        sel = (io == first).astype(jnp.float32)
        xk = x_ref[pl.ds(k2 * bk, bk), :]
        return acc + jnp.dot(sel, xk, preferred_element_type=jnp.float32)
    nx = lax.fori_loop(0, NP // bk, gath_step, jnp.zeros((G, D), jnp.float32))

    o_ref[...] = jnp.maximum(
        jnp.dot(nx, w_ref[...], preferred_element_type=jnp.float32)
        + bias_ref[...], 0.0)


@jax.jit
def _news(xp, b2, l0w, l0b):
    return pl.pallas_call(
        _news_body,
        out_shape=jax.ShapeDtypeStruct((G, H), jnp.float32),
    )(xp, b2, l0w, l0b)


KB = 40


def _sc_deg_body(dst_hbm, degp_hbm, vm_dst, vm_deg, vm_ones, sh_deg, sem_a):
    c = lax.axis_index("c")
    s = lax.axis_index("s")
    iota = lax.iota(jnp.int32, L)
    zero16 = jnp.zeros((L,), jnp.float32)
    r0 = s * RPS

    def z_deg(i, _):
        plsc.store_scatter(vm_deg, [iota + i * L], zero16)
        return 0
    lax.fori_loop(0, RPS // L, z_deg, 0)
    for jj in range(CH // L):
        vm_ones[pl.ds(jj * L, L)] = jnp.ones((L,), jnp.float32)
    pltpu.sync_copy(vm_deg, sh_deg.at[pl.ds(r0, RPS)])
    plsc.subcore_barrier()

    base_row = (c * NS + s) * KT

    def deg_batch(bb, _):
        pltpu.sync_copy(dst_hbm.at[pl.ds(base_row + bb * KB, KB)], vm_dst)

        def deg_grp(k, _2):
            descs = []
            for jj in range(8):
                descs.append(pltpu.async_copy(
                    vm_ones, sh_deg.at[vm_dst.at[k * 8 + jj]], sem_a,
                    add=True))
            for dsc in descs:
                dsc.wait()
            return 0
        lax.fori_loop(0, KB // 8, deg_grp, 0)
        return 0
    lax.fori_loop(0, KT // KB, deg_batch, 0)
    plsc.subcore_barrier()
    pltpu.sync_copy(sh_deg.at[pl.ds(r0, RPS)],
                    degp_hbm.at[c].at[pl.ds(r0, RPS)])


@functools.cache
def _get_sc_deg():
    return functools.partial(
        pl.kernel,
        out_type=jax.ShapeDtypeStruct((NC, NP), jnp.float32),
        mesh=plsc.VectorSubcoreMesh(core_axis_name="c", subcore_axis_name="s"),
        compiler_params=pltpu.CompilerParams(needs_layout_passes=False),
        scratch_types=[
            pltpu.VMEM((KB, CH), jnp.int32),
            pltpu.VMEM((RPS,), jnp.float32),
            pltpu.VMEM((CH,), jnp.float32),
            pltpu.VMEM_SHARED((NP,), jnp.float32),
            pltpu.SemaphoreType.DMA,
        ],
    )(_sc_deg_body)


def _scale_body(d0_ref, d1_ref, xw_ref, y_ref, dv_ref):
    deg = d0_ref[...] + d1_ref[...] + 1.0
    dv = lax.rsqrt(deg)
    dv_ref[...] = dv
    y_ref[...] = xw_ref[...] * dv


@jax.jit
def _scale(d0, d1, xw):
    bm = 512
    return pl.pallas_call(
        _scale_body,
        grid=(NP // bm,),
        in_specs=[
            pl.BlockSpec((bm, 1), lambda i: (i, 0)),
            pl.BlockSpec((bm, 1), lambda i: (i, 0)),
            pl.BlockSpec((bm, H), lambda i: (i, 0)),
        ],
        out_specs=[
            pl.BlockSpec((bm, H), lambda i: (i, 0)),
            pl.BlockSpec((bm, 1), lambda i: (i, 0)),
        ],
        out_shape=[
            jax.ShapeDtypeStruct((NP, H), jnp.float32),
            jax.ShapeDtypeStruct((NP, 1), jnp.float32),
        ],
    )(d0, d1, xw)


CH4 = 64
KB4 = 20


def _sc_agg_body(y_hbm, src_hbm, dst_hbm, convp_hbm,
                 vm_src, vm_dst, vm_r0, vm_r1, vm_r2, vm_r3,
                 sh_conv, sg0, sg1, sg2, sg3, ss0, ss1, ss2, ss3):
    c = lax.axis_index("c")
    s = lax.axis_index("s")
    iota = lax.iota(jnp.int32, L)
    zero16 = jnp.zeros((L,), jnp.float32)
    r0 = s * RPS
    bufs = (vm_r0, vm_r1, vm_r2, vm_r3)
    gsems = (sg0, sg1, sg2, sg3)
    ssems = (ss0, ss1, ss2, ss3)

    def z_row(i, _):
        ri = jnp.full((L,), i, jnp.int32)
        for jj in range(H // L):
            plsc.store_scatter(vm_r0, [ri, iota + jj * L], zero16)
        return 0
    lax.fori_loop(0, CH4, z_row, 0)
    for q in range(RPS // CH4):
        pltpu.sync_copy(vm_r0, sh_conv.at[pl.ds(r0 + q * CH4, CH4)])
    plsc.subcore_barrier()

    base_row = c * (RT // 2) + s * KT

    def agg_batch(bb, _):
        pltpu.sync_copy(src_hbm.at[pl.ds(base_row + bb * KB4, KB4)], vm_src)
        pltpu.sync_copy(dst_hbm.at[pl.ds(base_row + bb * KB4, KB4)], vm_dst)
        for r in range(4):
            pltpu.async_copy(y_hbm.at[vm_src.at[r // 2, r % 2]],
                             bufs[r], gsems[r])

        def group(k4, _2):
            for r in range(4):
                m = k4 * 4 + r
                j = m // 2
                h = m % 2
                pltpu.make_async_copy(y_hbm.at[vm_src.at[j, h]],
                                      bufs[r], gsems[r]).wait()
                pltpu.async_copy(bufs[r], sh_conv.at[vm_dst.at[j, h]],
                                 ssems[r], add=True)
            for r in range(4):
                m = k4 * 4 + r
                j = m // 2
                h = m % 2
                pltpu.make_async_copy(bufs[r], sh_conv.at[vm_dst.at[j, h]],
                                      ssems[r]).wait()

                @pl.when(m + 4 < KB4 * 2)
                def _():
                    j2 = (m + 4) // 2
                    h2 = (m + 4) % 2
                    pltpu.async_copy(y_hbm.at[vm_src.at[j2, h2]],
                                     bufs[r], gsems[r])
            return 0
        lax.fori_loop(0, KB4 // 2, group, 0)
        return 0
    lax.fori_loop(0, KT // KB4, agg_batch, 0)
    plsc.subcore_barrier()

    pltpu.sync_copy(sh_conv.at[pl.ds(r0, RPS)],
                    convp_hbm.at[c].at[pl.ds(r0, RPS)])


@functools.cache
def _get_sc_agg():
    return functools.partial(
        pl.kernel,
        out_type=jax.ShapeDtypeStruct((NC, NP, H), jnp.float32),
        mesh=plsc.VectorSubcoreMesh(core_axis_name="c", subcore_axis_name="s"),
        compiler_params=pltpu.CompilerParams(needs_layout_passes=False),
        scratch_types=[
            pltpu.VMEM((KB4, 2, CH4), jnp.int32),
            pltpu.VMEM((KB4, 2, CH4), jnp.int32),
            pltpu.VMEM((CH4, H), jnp.float32),
            pltpu.VMEM((CH4, H), jnp.float32),
            pltpu.VMEM((CH4, H), jnp.float32),
            pltpu.VMEM((CH4, H), jnp.float32),
            pltpu.VMEM_SHARED((NP, H), jnp.float32),
            pltpu.SemaphoreType.DMA,
            pltpu.SemaphoreType.DMA,
            pltpu.SemaphoreType.DMA,
            pltpu.SemaphoreType.DMA,
            pltpu.SemaphoreType.DMA,
            pltpu.SemaphoreType.DMA,
            pltpu.SemaphoreType.DMA,
            pltpu.SemaphoreType.DMA,
        ],
    )(_sc_agg_body)


def _sc_pool_body(convp_hbm, xw_hbm, dinv_hbm, batch_hbm, bias_hbm, out_hbm,
                  vm_batch, vm_dv, vm_p0, vm_p1, vm_xw, vm_b, vm_acc):
    c = lax.axis_index("c")
    s = lax.axis_index("s")
    w = s * NC + c
    iota = lax.iota(jnp.int32, L)
    ninf = jnp.full((L,), -jnp.inf, jnp.float32)

    def ini(i, _):
        ri = jnp.full((L,), i, jnp.int32)
        for jj in range(H // L):
            plsc.store_scatter(vm_acc, [ri, iota + jj * L], ninf)
        return 0
    lax.fori_loop(0, G, ini, 0)

    r0 = w * RPT
    base = pl.multiple_of((r0 // 128) * 128, 128)
    off = r0 - base
    pltpu.sync_copy(batch_hbm.at[pl.ds(base, RPT + 64)], vm_batch)
    pltpu.sync_copy(dinv_hbm.at[pl.ds(base, RPT + 64)], vm_dv)
    pltpu.sync_copy(bias_hbm, vm_b)
    bvs = [vm_b[pl.ds(jj * L, L)] for jj in range(H // L)]

    g0 = plsc.load_gather(vm_batch, [jnp.full((L,), off, jnp.int32)])[0]
    carry0 = (g0,) + tuple(ninf for _ in range(H // L))

    def chunk(cc2, carry):
        pltpu.sync_copy(convp_hbm.at[0].at[pl.ds(r0 + cc2 * 160, 160)], vm_p0)
        pltpu.sync_copy(convp_hbm.at[1].at[pl.ds(r0 + cc2 * 160, 160)], vm_p1)
        pltpu.sync_copy(xw_hbm.at[pl.ds(r0 + cc2 * 160, 160)], vm_xw)

        def row(i, cr):
            gcur = cr[0]
            ms = cr[1:]
            li = cc2 * 160 + i
            lii = jnp.full((L,), off + li, jnp.int32)
            bi = plsc.load_gather(vm_batch, [lii])[0]
            dvv = plsc.load_gather(vm_dv, [lii])
            dvv2 = dvv * dvv
            valid = jnp.where(r0 + li < N, 1.0, 0.0)
            vmask = jnp.full((L,), valid)
            ri = jnp.full((L,), i, jnp.int32)
            vs = []
            for jj in range(H // L):
                cidx = iota + jj * L
                p = (plsc.load_gather(vm_p0, [ri, cidx])
                     + plsc.load_gather(vm_p1, [ri, cidx]))
                xwv = plsc.load_gather(vm_xw, [ri, cidx])
                v = jnp.maximum(p * dvv + xwv * dvv2 + bvs[jj], 0.0) * vmask
                vs.append(v)

            def flush(ops):
                gold, mts, vts, bnew = ops
                gi = jnp.full((L,), gold, jnp.int32)
                for jj in range(H // L):
                    plsc.store_scatter(vm_acc, [gi, iota + jj * L], mts[jj])
                return (bnew,) + tuple(vts)

            def keep(ops):
                gold, mts, vts, _bnew = ops
                return (gold,) + tuple(
                    jnp.maximum(mts[jj], vts[jj]) for jj in range(H // L))

            return lax.cond(bi != gcur, flush, keep,
                            (gcur, tuple(ms), tuple(vs), bi))
        return lax.fori_loop(0, 160, row, carry)

    carry = lax.fori_loop(0, RPT // 160, chunk, carry0)
    gi = jnp.full((L,), carry[0], jnp.int32)
    for jj in range(H // L):
        plsc.store_scatter(vm_acc, [gi, iota + jj * L], carry[jj + 1])
    pltpu.sync_copy(vm_acc, out_hbm.at[w])


@functools.cache
def _get_sc_pool():
    return functools.partial(
        pl.kernel,
        out_type=jax.ShapeDtypeStruct((NW, G, H), jnp.float32),
        mesh=plsc.VectorSubcoreMesh(core_axis_name="c", subcore_axis_name="s"),
        compiler_params=pltpu.CompilerParams(needs_layout_passes=False),
        scratch_types=[
            pltpu.VMEM((RPT + 64,), jnp.int32),
            pltpu.VMEM((RPT + 64,), jnp.float32),
            pltpu.VMEM((160, H), jnp.float32),
            pltpu.VMEM((160, H), jnp.float32),
            pltpu.VMEM((160, H), jnp.float32),
            pltpu.VMEM((H,), jnp.float32),
            pltpu.VMEM((G, H), jnp.float32),
        ],
    )(_sc_pool_body)


def _final_body(pp_ref, news_ref, l1a_ref, l1b_ref, b1_ref, l2_ref, b2_ref,
                o_ref):
    def mx_step(k, acc):
        return jnp.maximum(acc, pp_ref[k])
    pooled = lax.fori_loop(0, NW, mx_step,
                           jnp.full((G, H), -jnp.inf, jnp.float32))
    h = jnp.dot(pooled, l1a_ref[...], preferred_element_type=jnp.float32)
    h = h + jnp.dot(news_ref[...], l1b_ref[...],
                    preferred_element_type=jnp.float32)
    h = jnp.maximum(h + b1_ref[...], 0.0)
    logits = jnp.dot(h, l2_ref[...], preferred_element_type=jnp.float32) \
        + b2_ref[...]
    m = jnp.max(logits, axis=-1, keepdims=True)
    lse = jnp.log(jnp.sum(jnp.exp(logits - m), axis=-1, keepdims=True)) + m
    o_ref[...] = logits - lse


@jax.jit
def _final(pp, news, l1a, l1b, b1, l2p, b2p):
    return pl.pallas_call(
        _final_body,
        out_shape=jax.ShapeDtypeStruct((G, H), jnp.float32),
    )(pp, news, l1a, l1b, b1, l2p, b2p)


@jax.jit
def kernel(x, edge_index, batch, W, b, lin0_W, lin0_b, lin1_W, lin1_b,
           lin2_W, lin2_b):
    x = x.astype(jnp.float32)
    xp = jnp.pad(x, ((0, NP - N), (0, 0)))
    src = edge_index[0].astype(jnp.int32)
    dst = edge_index[1].astype(jnp.int32)
    ep = RT * CH
    srcp = jnp.concatenate(
        [src, jnp.zeros((ep - E,), jnp.int32)]).reshape(RT, CH)
    dstp = jnp.concatenate(
        [dst, jnp.full((ep - E,), TRASH, jnp.int32)]).reshape(RT, CH)
    batchp = jnp.concatenate(
        [batch.astype(jnp.int32), jnp.full((NP - N,), G - 1, jnp.int32)])
    b2 = batchp.reshape(NP // 128, 128)

    xw = _matmul(xp, W)
    news = _news(xp, b2, lin0_W, lin0_b.reshape(1, H))
    degp = _get_sc_deg()(dstp)
    y, dinvc = _scale(degp[0].reshape(NP, 1), degp[1].reshape(NP, 1), xw)
    convp = _get_sc_agg()(y, srcp.reshape(RT, 2, 64),
                          dstp.reshape(RT, 2, 64))
    pooled_part = _get_sc_pool()(convp, xw, dinvc.reshape(NP), batchp, b)

    l2p = jnp.pad(lin2_W, ((0, 0), (0, H - OUTD)))
    b2p = jnp.concatenate(
        [lin2_b, jnp.full((H - OUTD,), -1e30, jnp.float32)]).reshape(1, H)
    out = _final(pooled_part, news, lin1_W[:H], lin1_W[H:],
                 lin1_b.reshape(1, H), l2p, b2p)
    return out[:, :OUTD]


# speedup vs baseline: 1.0446x; 1.0446x over previous
import functools

import jax
import jax.numpy as jnp
from jax import lax
from jax.experimental import pallas as pl
from jax.experimental.pallas import tpu as pltpu
from jax.experimental.pallas import tpu_sc as plsc

N = 10000
NP = 10240
E = 320000
D = 128
H = 128
G = 128
OUTD = 2
NC = 2
NS = 16
NW = NC * NS
L = 16
CH = 128
KT = 80
RT = NW * KT
KD = RT // NS
RPT = NP // NW
RPS = NP // NS
TRASH = N


def _mm_body(x_ref, w_ref, o_ref):
    o_ref[...] = jnp.dot(x_ref[...], w_ref[...], preferred_element_type=jnp.float32)


@jax.jit
def _matmul(x, w):
    m, k = x.shape
    n = w.shape[1]
    bm = 512
    return pl.pallas_call(
        _mm_body,
        grid=(m // bm,),
        in_specs=[
            pl.BlockSpec((bm, k), lambda i: (i, 0)),
            pl.BlockSpec((k, n), lambda i: (0, 0)),
        ],
        out_specs=pl.BlockSpec((bm, n), lambda i: (i, 0)),
        out_shape=jax.ShapeDtypeStruct((m, n), jnp.float32),
    )(x, w)


def _news_body(x_ref, b2_ref, w_ref, bias_ref, o_ref):
    rows_g = lax.broadcasted_iota(jnp.int32, (G, 1), 0)

    def cnt_step(k2, acc):
        blk8 = b2_ref[pl.ds(k2 * 8, 8), :]
        for r8 in range(8):
            row = blk8[r8:r8 + 1, :]
            acc = acc + jnp.sum((rows_g == row).astype(jnp.float32), axis=1,
                                keepdims=True)
        return acc
    cnt = lax.fori_loop(0, (NP // 128) // 8, cnt_step,
                        jnp.zeros((G, 1), jnp.float32))

    rr = lax.broadcasted_iota(jnp.int32, (G, G), 0)
    cc = lax.broadcasted_iota(jnp.int32, (G, G), 1)
    m_lt = (cc < rr).astype(jnp.float32)
    first = jnp.dot(m_lt, cnt, preferred_element_type=jnp.float32)
    first = jnp.minimum(first, float(N - 1)).astype(jnp.int32)

    bk = 256

    def gath_step(k2, acc):
        io = lax.broadcasted_iota(jnp.int32, (G, bk), 1) + k2 * bk
        sel = (io == first).astype(jnp.float32)
        xk = x_ref[pl.ds(k2 * bk, bk), :]
        return acc + jnp.dot(sel, xk, preferred_element_type=jnp.float32)
    nx = lax.fori_loop(0, NP // bk, gath_step, jnp.zeros((G, D), jnp.float32))

    o_ref[...] = jnp.maximum(
        jnp.dot(nx, w_ref[...], preferred_element_type=jnp.float32)
        + bias_ref[...], 0.0)


@jax.jit
def _news(xp, b2, l0w, l0b):
    return pl.pallas_call(
        _news_body,
        out_shape=jax.ShapeDtypeStruct((G, H), jnp.float32),
    )(xp, b2, l0w, l0b)


KB = 40


def _sc_deg_body(dst_hbm, degp_hbm, vm_dst, vm_deg, vm_ones, sh_deg, sem_a):
    c = lax.axis_index("c")
    s = lax.axis_index("s")
    iota = lax.iota(jnp.int32, L)
    zero16 = jnp.zeros((L,), jnp.float32)
    r0 = s * RPS

    def z_deg(i, _):
        plsc.store_scatter(vm_deg, [iota + i * L], zero16)
        return 0
    lax.fori_loop(0, RPS // L, z_deg, 0)
    for jj in range(CH // L):
        vm_ones[pl.ds(jj * L, L)] = jnp.ones((L,), jnp.float32)
    pltpu.sync_copy(vm_deg, sh_deg.at[pl.ds(r0, RPS)])
    plsc.subcore_barrier()

    base_row = (c * NS + s) * KT

    def deg_batch(bb, _):
        pltpu.sync_copy(dst_hbm.at[pl.ds(base_row + bb * KB, KB)], vm_dst)

        def deg_grp(k, _2):
            descs = []
            for jj in range(8):
                descs.append(pltpu.async_copy(
                    vm_ones, sh_deg.at[vm_dst.at[k * 8 + jj]], sem_a,
                    add=True))
            for dsc in descs:
                dsc.wait()
            return 0
        lax.fori_loop(0, KB // 8, deg_grp, 0)
        return 0
    lax.fori_loop(0, KT // KB, deg_batch, 0)
    plsc.subcore_barrier()
    pltpu.sync_copy(sh_deg.at[pl.ds(r0, RPS)],
                    degp_hbm.at[c].at[pl.ds(r0, RPS)])


@functools.cache
def _get_sc_deg():
    return functools.partial(
        pl.kernel,
        out_type=jax.ShapeDtypeStruct((NC, NP), jnp.float32),
        mesh=plsc.VectorSubcoreMesh(core_axis_name="c", subcore_axis_name="s"),
        compiler_params=pltpu.CompilerParams(needs_layout_passes=False),
        scratch_types=[
            pltpu.VMEM((KB, CH), jnp.int32),
            pltpu.VMEM((RPS,), jnp.float32),
            pltpu.VMEM((CH,), jnp.float32),
            pltpu.VMEM_SHARED((NP,), jnp.float32),
            pltpu.SemaphoreType.DMA,
        ],
    )(_sc_deg_body)


def _scale_body(d0_ref, d1_ref, xw_ref, y_ref, dv_ref):
    deg = d0_ref[...] + d1_ref[...] + 1.0
    dv = lax.rsqrt(deg)
    dv_ref[...] = dv
    y_ref[...] = xw_ref[...] * dv


@jax.jit
def _scale(d0, d1, xw):
    bm = 512
    return pl.pallas_call(
        _scale_body,
        grid=(NP // bm,),
        in_specs=[
            pl.BlockSpec((bm, 1), lambda i: (i, 0)),
            pl.BlockSpec((bm, 1), lambda i: (i, 0)),
            pl.BlockSpec((bm, H), lambda i: (i, 0)),
        ],
        out_specs=[
            pl.BlockSpec((bm, H), lambda i: (i, 0)),
            pl.BlockSpec((bm, 1), lambda i: (i, 0)),
        ],
        out_shape=[
            jax.ShapeDtypeStruct((NP, H), jnp.float32),
            jax.ShapeDtypeStruct((NP, 1), jnp.float32),
        ],
    )(d0, d1, xw)


def _sc_agg_body(y_hbm, src_hbm, dst_hbm, convp_hbm,
                 vm_src, vm_dst, vm_ra, vm_rb,
                 sh_conv, sem_a, sem_b, sem_sa, sem_sb):
    c = lax.axis_index("c")
    yc = y_hbm
    s = lax.axis_index("s")
    iota = lax.iota(jnp.int32, L)
    zero16 = jnp.zeros((L,), jnp.float32)
    r0 = s * RPS

    def z_row(i, _):
        ri = jnp.full((L,), i, jnp.int32)
        for jj in range(H // L):
            plsc.store_scatter(vm_ra, [ri, iota + jj * L], zero16)
        return 0
    lax.fori_loop(0, 128, z_row, 0)
    for q in range(RPS // 128):
        pltpu.sync_copy(vm_ra, sh_conv.at[pl.ds(r0 + q * 128, 128)])
    plsc.subcore_barrier()

    base_row = c * (RT // 2) + s * KT
    nbatch = KT // KB

    def agg_batch(bb, _):
        pltpu.sync_copy(src_hbm.at[pl.ds(base_row + bb * KB, KB)], vm_src)
        pltpu.sync_copy(dst_hbm.at[pl.ds(base_row + bb * KB, KB)], vm_dst)
        pltpu.async_copy(yc.at[vm_src.at[0]], vm_ra, sem_a)
        pltpu.async_copy(yc.at[vm_src.at[1]], vm_rb, sem_b)

        def pair(k, _):
            j0 = k * 2
            j1 = j0 + 1
            pltpu.make_async_copy(yc.at[vm_src.at[j0]], vm_ra, sem_a).wait()
            pltpu.async_copy(vm_ra, sh_conv.at[vm_dst.at[j0]], sem_sa,
                             add=True)
            pltpu.make_async_copy(yc.at[vm_src.at[j1]], vm_rb, sem_b).wait()
            pltpu.async_copy(vm_rb, sh_conv.at[vm_dst.at[j1]], sem_sb,
                             add=True)
            pltpu.make_async_copy(vm_ra, sh_conv.at[vm_dst.at[j0]],
                                  sem_sa).wait()

            @pl.when(j0 + 2 < KB)
            def _():
                pltpu.async_copy(yc.at[vm_src.at[j0 + 2]], vm_ra, sem_a)
            pltpu.make_async_copy(vm_rb, sh_conv.at[vm_dst.at[j1]],
                                  sem_sb).wait()

            @pl.when(j1 + 2 < KB)
            def _():
                pltpu.async_copy(yc.at[vm_src.at[j1 + 2]], vm_rb, sem_b)
            return 0
        lax.fori_loop(0, KB // 2, pair, 0)
        return 0
    lax.fori_loop(0, nbatch, agg_batch, 0)
    plsc.subcore_barrier()

    pltpu.sync_copy(sh_conv.at[pl.ds(r0, RPS)],
                    convp_hbm.at[c].at[pl.ds(r0, RPS)])


@functools.cache
def _get_sc_agg():
    return functools.partial(
        pl.kernel,
        out_type=jax.ShapeDtypeStruct((NC, NP, H), jnp.float32),
        mesh=plsc.VectorSubcoreMesh(core_axis_name="c", subcore_axis_name="s"),
        compiler_params=pltpu.CompilerParams(needs_layout_passes=False),
        scratch_types=[
            pltpu.VMEM((KB, CH), jnp.int32),
            pltpu.VMEM((KB, CH), jnp.int32),
            pltpu.VMEM((CH, H), jnp.float32),
            pltpu.VMEM((CH, H), jnp.float32),
            pltpu.VMEM_SHARED((NP, H), jnp.float32),
            pltpu.SemaphoreType.DMA,
            pltpu.SemaphoreType.DMA,
            pltpu.SemaphoreType.DMA,
            pltpu.SemaphoreType.DMA,
        ],
    )(_sc_agg_body)


def _sc_pool_body(convp_hbm, xw_hbm, dinv_hbm, batch_hbm, bias_hbm, out_hbm,
                  vm_batch, vm_dv, vm_p0, vm_p1, vm_xw, vm_b, vm_acc):
    c = lax.axis_index("c")
    s = lax.axis_index("s")
    w = s * NC + c
    iota = lax.iota(jnp.int32, L)
    ninf = jnp.full((L,), -jnp.inf, jnp.float32)

    def ini(i, _):
        ri = jnp.full((L,), i, jnp.int32)
        for jj in range(H // L):
            plsc.store_scatter(vm_acc, [ri, iota + jj * L], ninf)
        return 0
    lax.fori_loop(0, G, ini, 0)

    r0 = w * RPT
    base = pl.multiple_of((r0 // 128) * 128, 128)
    off = r0 - base
    pltpu.sync_copy(batch_hbm.at[pl.ds(base, RPT + 64)], vm_batch)
    pltpu.sync_copy(dinv_hbm.at[pl.ds(base, RPT + 64)], vm_dv)
    pltpu.sync_copy(bias_hbm, vm_b)
    bvs = [vm_b[pl.ds(jj * L, L)] for jj in range(H // L)]

    g0 = plsc.load_gather(vm_batch, [jnp.full((L,), off, jnp.int32)])[0]
    carry0 = (g0,) + tuple(ninf for _ in range(H // L))

    def chunk(cc2, carry):
        pltpu.sync_copy(convp_hbm.at[0].at[pl.ds(r0 + cc2 * 160, 160)], vm_p0)
        pltpu.sync_copy(convp_hbm.at[1].at[pl.ds(r0 + cc2 * 160, 160)], vm_p1)
        pltpu.sync_copy(xw_hbm.at[pl.ds(r0 + cc2 * 160, 160)], vm_xw)

        def row(i, cr):
            gcur = cr[0]
            ms = cr[1:]
            li = cc2 * 160 + i
            lii = jnp.full((L,), off + li, jnp.int32)
            bi = plsc.load_gather(vm_batch, [lii])[0]
            dvv = plsc.load_gather(vm_dv, [lii])
            dvv2 = dvv * dvv
            valid = jnp.where(r0 + li < N, 1.0, 0.0)
            vmask = jnp.full((L,), valid)
            ri = jnp.full((L,), i, jnp.int32)
            vs = []
            for jj in range(H // L):
                cidx = iota + jj * L
                p = (plsc.load_gather(vm_p0, [ri, cidx])
                     + plsc.load_gather(vm_p1, [ri, cidx]))
                xwv = plsc.load_gather(vm_xw, [ri, cidx])
                v = jnp.maximum(p * dvv + xwv * dvv2 + bvs[jj], 0.0) * vmask
                vs.append(v)

            def flush(ops):
                gold, mts, vts, bnew = ops
                gi = jnp.full((L,), gold, jnp.int32)
                for jj in range(H // L):
                    plsc.store_scatter(vm_acc, [gi, iota + jj * L], mts[jj])
                return (bnew,) + tuple(vts)

            def keep(ops):
                gold, mts, vts, _bnew = ops
                return (gold,) + tuple(
                    jnp.maximum(mts[jj], vts[jj]) for jj in range(H // L))

            return lax.cond(bi != gcur, flush, keep,
                            (gcur, tuple(ms), tuple(vs), bi))
        return lax.fori_loop(0, 160, row, carry)

    carry = lax.fori_loop(0, RPT // 160, chunk, carry0)
    gi = jnp.full((L,), carry[0], jnp.int32)
    for jj in range(H // L):
        plsc.store_scatter(vm_acc, [gi, iota + jj * L], carry[jj + 1])
    pltpu.sync_copy(vm_acc, out_hbm.at[w])


@functools.cache
def _get_sc_pool():
    return functools.partial(
        pl.kernel,
        out_type=jax.ShapeDtypeStruct((NW, G, H), jnp.float32),
        mesh=plsc.VectorSubcoreMesh(core_axis_name="c", subcore_axis_name="s"),
        compiler_params=pltpu.CompilerParams(needs_layout_passes=False),
        scratch_types=[
            pltpu.VMEM((RPT + 64,), jnp.int32),
            pltpu.VMEM((RPT + 64,), jnp.float32),
            pltpu.VMEM((160, H), jnp.float32),
            pltpu.VMEM((160, H), jnp.float32),
            pltpu.VMEM((160, H), jnp.float32),
            pltpu.VMEM((H,), jnp.float32),
            pltpu.VMEM((G, H), jnp.float32),
        ],
    )(_sc_pool_body)


def _final_body(pp_ref, news_ref, l1a_ref, l1b_ref, b1_ref, l2_ref, b2_ref,
                o_ref):
    def mx_step(k, acc):
        return jnp.maximum(acc, pp_ref[k])
    pooled = lax.fori_loop(0, NW, mx_step,
                           jnp.full((G, H), -jnp.inf, jnp.float32))
    h = jnp.dot(pooled, l1a_ref[...], preferred_element_type=jnp.float32)
    h = h + jnp.dot(news_ref[...], l1b_ref[...],
                    preferred_element_type=jnp.float32)
    h = jnp.maximum(h + b1_ref[...], 0.0)
    logits = jnp.dot(h, l2_ref[...], preferred_element_type=jnp.float32) \
        + b2_ref[...]
    m = jnp.max(logits, axis=-1, keepdims=True)
    lse = jnp.log(jnp.sum(jnp.exp(logits - m), axis=-1, keepdims=True)) + m
    o_ref[...] = logits - lse


@jax.jit
def _final(pp, news, l1a, l1b, b1, l2p, b2p):
    return pl.pallas_call(
        _final_body,
        out_shape=jax.ShapeDtypeStruct((G, H), jnp.float32),
    )(pp, news, l1a, l1b, b1, l2p, b2p)


@jax.jit
def kernel(x, edge_index, batch, W, b, lin0_W, lin0_b, lin1_W, lin1_b,
           lin2_W, lin2_b):
    x = x.astype(jnp.float32)
    xp = jnp.pad(x, ((0, NP - N), (0, 0)))
    src = edge_index[0].astype(jnp.int32)
    dst = edge_index[1].astype(jnp.int32)
    ep = RT * CH
    srcp = jnp.concatenate(
        [src, jnp.zeros((ep - E,), jnp.int32)]).reshape(RT, CH)
    dstp = jnp.concatenate(
        [dst, jnp.full((ep - E,), TRASH, jnp.int32)]).reshape(RT, CH)
    batchp = jnp.concatenate(
        [batch.astype(jnp.int32), jnp.full((NP - N,), G - 1, jnp.int32)])
    b2 = batchp.reshape(NP // 128, 128)

    xw = _matmul(xp, W)
    news = _news(xp, b2, lin0_W, lin0_b.reshape(1, H))
    degp = _get_sc_deg()(dstp)
    y, dinvc = _scale(degp[0].reshape(NP, 1), degp[1].reshape(NP, 1), xw)
    convp = _get_sc_agg()(y, srcp, dstp)
    pooled_part = _get_sc_pool()(convp, xw, dinvc.reshape(NP), batchp, b)

    l2p = jnp.pad(lin2_W, ((0, 0), (0, H - OUTD)))
    b2p = jnp.concatenate(
        [lin2_b, jnp.full((H - OUTD,), -1e30, jnp.float32)]).reshape(1, H)
    out = _final(pooled_part, news, lin1_W[:H], lin1_W[H:],
                 lin1_b.reshape(1, H), l2p, b2p)
    return out[:, :OUTD]
